# speedup vs baseline: 11.8898x; 11.8898x over previous
import jax
import jax.numpy as jnp
from jax.experimental import pallas as pl

N = 50000
E = 800000
EMBED = 64
HEADS = 8
HEAD_DIM = EMBED // HEADS


def _ln(x, g, b, eps=1e-5):
    m = x.mean(-1, keepdims=True)
    v = ((x - m) ** 2).mean(-1, keepdims=True)
    return (x - m) / jnp.sqrt(v + eps) * g + b


def _lin(x, w, b):
    return x @ w + b


def kernel(x, t, edge_index, edge_attr, bos_mask, params):
    p = params
    cp = p['center']
    h = jax.nn.relu(_ln(_lin(x, cp['w1'], cp['b1']), cp['g1'], cp['be1']))
    h = jax.nn.relu(_ln(_lin(h, cp['w2'], cp['b2']), cp['g2'], cp['be2']))
    center = _ln(_lin(h, cp['w3'], cp['b3']), cp['g3'], cp['be3'])
    center = jnp.where(bos_mask[:, None], p['bos_token'][t], center)
    ce_n = _ln(center, p['norm1']['g'], p['norm1']['b'])
    qn = _lin(ce_n, p['lin_q']['w'], p['lin_q']['b'])
    b0 = p['nbr']['branch0']
    hb = jax.nn.relu(_ln(_lin(x, b0['w1'], b0['b1']), b0['g1'], b0['be1']))
    b0n = _lin(hb, b0['w2'], b0['b2'])

    src = edge_index[0]
    dst = edge_index[1]
    b1 = p['nbr']['branch1']
    h1 = jax.nn.relu(_ln(_lin(edge_attr, b1['w1'], b1['b1']), b1['g1'], b1['be1']))
    s = b0n[src] + _lin(h1, b1['w2'], b1['b2'])
    a = p['nbr']['aggr']
    hn = jax.nn.relu(_ln(s, a['g1'], a['be1']))
    nbr = _ln(_lin(hn, a['w'], a['b']), a['g2'], a['be2'])
    k = _lin(nbr, p['lin_k']['w'], p['lin_k']['b']).reshape(-1, HEADS, HEAD_DIM)
    v = _lin(nbr, p['lin_v']['w'], p['lin_v']['b']).reshape(-1, HEADS, HEAD_DIM)
    q = qn[dst].reshape(-1, HEADS, HEAD_DIM)
    alpha = (q * k).sum(-1) / (HEAD_DIM ** 0.5)
    ex = jnp.exp(alpha)
    denom = jax.ops.segment_sum(ex, dst, num_segments=N)
    num = jax.ops.segment_sum((v * ex[..., None]).reshape(-1, EMBED), dst,
                              num_segments=N)
    agg = num / jnp.repeat(denom + 1e-16, HEAD_DIM, axis=-1)

    gate = jax.nn.sigmoid(_lin(agg, p['lin_ih']['w'], p['lin_ih']['b']) +
                          _lin(ce_n, p['lin_hh']['w'], p['lin_hh']['b']))
    upd = agg + gate * (_lin(ce_n, p['lin_self']['w'], p['lin_self']['b']) - agg)
    center = center + _lin(upd, p['out_proj']['w'], p['out_proj']['b'])
    h = _ln(center, p['norm2']['g'], p['norm2']['b'])
    h = jax.nn.relu(_lin(h, p['mlp']['w1'], p['mlp']['b1']))
    center = center + _lin(h, p['mlp']['w2'], p['mlp']['b2'])
    return center


# speedup vs baseline: 13.0267x; 1.0956x over previous
import functools

import jax
import jax.numpy as jnp
from jax import lax
from jax.experimental import pallas as pl
from jax.experimental.pallas import tpu as pltpu
from jax.experimental.pallas import tpu_sc as plsc

N = 50000
E = 800000
EMBED = 64
HEADS = 8
HEAD_DIM = EMBED // HEADS

_NW = 32
_PER_W = E // _NW
_GCH = 200


def _sc_gather(tab, src, dst):
    mesh = plsc.VectorSubcoreMesh(core_axis_name="c", subcore_axis_name="s")

    @functools.partial(
        pl.kernel,
        out_type=(jax.ShapeDtypeStruct((E, 128), jnp.float32),
                  jax.ShapeDtypeStruct((E, 128), jnp.float32)),
        mesh=mesh,
        scratch_types=[
            pltpu.VMEM((_GCH,), jnp.int32),
            pltpu.VMEM((_GCH,), jnp.int32),
            pltpu.VMEM((_GCH, 128), jnp.float32),
            pltpu.VMEM((_GCH, 128), jnp.float32),
            pltpu.SemaphoreType.DMA,
            pltpu.SemaphoreType.DMA,
        ],
    )
    def k(tab_hbm, src_hbm, dst_hbm, os_hbm, od_hbm, si, di, bb, qb, s0, s1):
        wid = lax.axis_index("s") * 2 + lax.axis_index("c")
        base = wid * _PER_W

        @pl.loop(0, _PER_W // _GCH)
        def _(i):
            off = base + i * _GCH
            pltpu.sync_copy(src_hbm.at[pl.ds(off, _GCH)], si)
            pltpu.sync_copy(dst_hbm.at[pl.ds(off, _GCH)], di)
            cb = pltpu.async_copy(tab_hbm.at[si], bb, s0)
            cq = pltpu.async_copy(tab_hbm.at[di], qb, s1)
            cb.wait()
            cq.wait()
            pltpu.sync_copy(bb, os_hbm.at[pl.ds(off, _GCH)])
            pltpu.sync_copy(qb, od_hbm.at[pl.ds(off, _GCH)])

    return k(tab, src, dst)


def _ln(x, g, b, eps=1e-5):
    m = x.mean(-1, keepdims=True)
    v = ((x - m) ** 2).mean(-1, keepdims=True)
    return (x - m) / jnp.sqrt(v + eps) * g + b


def _lin(x, w, b):
    return x @ w + b


def kernel(x, t, edge_index, edge_attr, bos_mask, params):
    p = params
    cp = p['center']
    h = jax.nn.relu(_ln(_lin(x, cp['w1'], cp['b1']), cp['g1'], cp['be1']))
    h = jax.nn.relu(_ln(_lin(h, cp['w2'], cp['b2']), cp['g2'], cp['be2']))
    center = _ln(_lin(h, cp['w3'], cp['b3']), cp['g3'], cp['be3'])
    center = jnp.where(bos_mask[:, None], p['bos_token'][t], center)
    ce_n = _ln(center, p['norm1']['g'], p['norm1']['b'])
    qn = _lin(ce_n, p['lin_q']['w'], p['lin_q']['b'])
    b0 = p['nbr']['branch0']
    hb = jax.nn.relu(_ln(_lin(x, b0['w1'], b0['b1']), b0['g1'], b0['be1']))
    b0n = _lin(hb, b0['w2'], b0['b2'])

    src = edge_index[0]
    dst = edge_index[1]
    b1 = p['nbr']['branch1']
    h1 = jax.nn.relu(_ln(_lin(edge_attr, b1['w1'], b1['b1']), b1['g1'], b1['be1']))
    tab = jnp.concatenate([b0n, qn], axis=1)
    os_, od_ = _sc_gather(tab, src, dst)
    bs, qd = os_[:, :64], od_[:, 64:]
    s = bs + _lin(h1, b1['w2'], b1['b2'])
    a = p['nbr']['aggr']
    hn = jax.nn.relu(_ln(s, a['g1'], a['be1']))
    nbr = _ln(_lin(hn, a['w'], a['b']), a['g2'], a['be2'])
    k = _lin(nbr, p['lin_k']['w'], p['lin_k']['b']).reshape(-1, HEADS, HEAD_DIM)
    v = _lin(nbr, p['lin_v']['w'], p['lin_v']['b']).reshape(-1, HEADS, HEAD_DIM)
    q = qd.reshape(-1, HEADS, HEAD_DIM)
    alpha = (q * k).sum(-1) / (HEAD_DIM ** 0.5)
    ex = jnp.exp(alpha)
    denom = jax.ops.segment_sum(ex, dst, num_segments=N)
    num = jax.ops.segment_sum((v * ex[..., None]).reshape(-1, EMBED), dst,
                              num_segments=N)
    agg = num / jnp.repeat(denom + 1e-16, HEAD_DIM, axis=-1)

    gate = jax.nn.sigmoid(_lin(agg, p['lin_ih']['w'], p['lin_ih']['b']) +
                          _lin(ce_n, p['lin_hh']['w'], p['lin_hh']['b']))
    upd = agg + gate * (_lin(ce_n, p['lin_self']['w'], p['lin_self']['b']) - agg)
    center = center + _lin(upd, p['out_proj']['w'], p['out_proj']['b'])
    h = _ln(center, p['norm2']['g'], p['norm2']['b'])
    h = jax.nn.relu(_lin(h, p['mlp']['w1'], p['mlp']['b1']))
    center = center + _lin(h, p['mlp']['w2'], p['mlp']['b2'])
    return center


# speedup vs baseline: 15.0000x; 1.1515x over previous
import functools

import jax
import jax.numpy as jnp
from jax import lax
from jax.experimental import pallas as pl
from jax.experimental.pallas import tpu as pltpu
from jax.experimental.pallas import tpu_sc as plsc

N = 50000
E = 800000
EMBED = 64
HEADS = 8
HEAD_DIM = EMBED // HEADS

_NW = 32
_PER_W = E // _NW
_GCH = 200


def _sc_gather(tab, src, dst):
    mesh = plsc.VectorSubcoreMesh(core_axis_name="c", subcore_axis_name="s")

    @functools.partial(
        pl.kernel,
        out_type=(jax.ShapeDtypeStruct((E, 128), jnp.float32),
                  jax.ShapeDtypeStruct((E, 128), jnp.float32)),
        mesh=mesh,
        scratch_types=[
            pltpu.VMEM((_GCH,), jnp.int32),
            pltpu.VMEM((_GCH,), jnp.int32),
            pltpu.VMEM((_GCH, 128), jnp.float32),
            pltpu.VMEM((_GCH, 128), jnp.float32),
            pltpu.SemaphoreType.DMA,
            pltpu.SemaphoreType.DMA,
        ],
    )
    def k(tab_hbm, src_hbm, dst_hbm, os_hbm, od_hbm, si, di, bb, qb, s0, s1):
        wid = lax.axis_index("s") * 2 + lax.axis_index("c")
        base = wid * _PER_W

        @pl.loop(0, _PER_W // _GCH)
        def _(i):
            off = base + i * _GCH
            pltpu.sync_copy(src_hbm.at[pl.ds(off, _GCH)], si)
            pltpu.sync_copy(dst_hbm.at[pl.ds(off, _GCH)], di)
            cb = pltpu.async_copy(tab_hbm.at[si], bb, s0)
            cq = pltpu.async_copy(tab_hbm.at[di], qb, s1)
            cb.wait()
            cq.wait()
            pltpu.sync_copy(bb, os_hbm.at[pl.ds(off, _GCH)])
            pltpu.sync_copy(qb, od_hbm.at[pl.ds(off, _GCH)])

    return k(tab, src, dst)


_SCH = 128
_SE_A = 50048
_SE_LAST = E - 15 * _SE_A
_Q = 12512
_AR = 12800


def _sc_scatter(payload, dst, zeros128):
    mesh = plsc.VectorSubcoreMesh(core_axis_name="c", subcore_axis_name="s")

    @functools.partial(
        pl.kernel,
        out_type=jax.ShapeDtypeStruct((4 * _Q, 128), jnp.float32),
        mesh=mesh,
        scratch_types=[
            pltpu.VMEM((_SCH, 128), jnp.float32),
            pltpu.VMEM((_SCH,), jnp.int32),
            pltpu.VMEM_SHARED((_AR, 128), jnp.float32),
            pltpu.SemaphoreType.DMA,
        ],
    )
    def k(p_hbm, dst_hbm, z_hbm, o_hbm, pay, idx, acc, sem):
        c = lax.axis_index("c")
        s = lax.axis_index("s")
        trash = _Q + s
        ebase = s * _SE_A

        for p in range(2):
            qbase = (2 * p + c) * _Q
            @pl.loop(0, 5)
            def _(j):
                pltpu.sync_copy(z_hbm, acc.at[pl.ds(s * 800 + j * 160, 160)])

            plsc.subcore_barrier()

            def run(nch):
                @pl.loop(0, nch)
                def _(i):
                    off = ebase + i * _SCH
                    pltpu.sync_copy(dst_hbm.at[pl.ds(off, _SCH)], idx)
                    pltpu.sync_copy(p_hbm.at[pl.ds(off, _SCH)], pay)

                    @pl.loop(0, _SCH // 16)
                    def _(j):
                        d = idx[pl.ds(j * 16, 16)] - qbase
                        ok = (d >= 0) & (d < _Q)
                        idx[pl.ds(j * 16, 16)] = jnp.where(ok, d, trash)

                    pltpu.sync_copy(pay, acc.at[idx], add=True)

            @pl.when(s < 15)
            def _():
                run(_SE_A // _SCH)

            @pl.when(s == 15)
            def _():
                run(_SE_LAST // _SCH)

            plsc.subcore_barrier()
            @pl.when(s < 15)
            def _():
                pltpu.sync_copy(acc.at[pl.ds(s * 784, 784)],
                                o_hbm.at[pl.ds(qbase + s * 784, 784)])

            @pl.when(s == 15)
            def _():
                pltpu.sync_copy(acc.at[pl.ds(15 * 784, _Q - 15 * 784)],
                                o_hbm.at[pl.ds(qbase + 15 * 784, _Q - 15 * 784)])

            plsc.subcore_barrier()

    return k(payload, dst, zeros128)


def _ln(x, g, b, eps=1e-5):
    m = x.mean(-1, keepdims=True)
    v = ((x - m) ** 2).mean(-1, keepdims=True)
    return (x - m) / jnp.sqrt(v + eps) * g + b


def _lin(x, w, b):
    return x @ w + b


def kernel(x, t, edge_index, edge_attr, bos_mask, params):
    p = params
    cp = p['center']
    h = jax.nn.relu(_ln(_lin(x, cp['w1'], cp['b1']), cp['g1'], cp['be1']))
    h = jax.nn.relu(_ln(_lin(h, cp['w2'], cp['b2']), cp['g2'], cp['be2']))
    center = _ln(_lin(h, cp['w3'], cp['b3']), cp['g3'], cp['be3'])
    center = jnp.where(bos_mask[:, None], p['bos_token'][t], center)
    ce_n = _ln(center, p['norm1']['g'], p['norm1']['b'])
    qn = _lin(ce_n, p['lin_q']['w'], p['lin_q']['b'])
    b0 = p['nbr']['branch0']
    hb = jax.nn.relu(_ln(_lin(x, b0['w1'], b0['b1']), b0['g1'], b0['be1']))
    b0n = _lin(hb, b0['w2'], b0['b2'])

    src = edge_index[0]
    dst = edge_index[1]
    b1 = p['nbr']['branch1']
    h1 = jax.nn.relu(_ln(_lin(edge_attr, b1['w1'], b1['b1']), b1['g1'], b1['be1']))
    tab = jnp.concatenate([b0n, qn], axis=1)
    os_, od_ = _sc_gather(tab, src, dst)
    bs, qd = os_[:, :64], od_[:, 64:]
    s = bs + _lin(h1, b1['w2'], b1['b2'])
    a = p['nbr']['aggr']
    hn = jax.nn.relu(_ln(s, a['g1'], a['be1']))
    nbr = _ln(_lin(hn, a['w'], a['b']), a['g2'], a['be2'])
    k = _lin(nbr, p['lin_k']['w'], p['lin_k']['b']).reshape(-1, HEADS, HEAD_DIM)
    v = _lin(nbr, p['lin_v']['w'], p['lin_v']['b']).reshape(-1, HEADS, HEAD_DIM)
    q = qd.reshape(-1, HEADS, HEAD_DIM)
    alpha = (q * k).sum(-1) / (HEAD_DIM ** 0.5)
    ex = jnp.exp(alpha)
    evx = (v * ex[..., None]).reshape(-1, EMBED)
    payload = jnp.concatenate(
        [ex, evx, jnp.zeros((E, 56), jnp.float32)], axis=1)
    zeros128 = jnp.zeros((160, 128), jnp.float32)
    o = _sc_scatter(payload, dst, zeros128)
    denom = o[:N, :8]
    num = o[:N, 8:72]
    agg = num / jnp.repeat(denom + 1e-16, HEAD_DIM, axis=-1)

    gate = jax.nn.sigmoid(_lin(agg, p['lin_ih']['w'], p['lin_ih']['b']) +
                          _lin(ce_n, p['lin_hh']['w'], p['lin_hh']['b']))
    upd = agg + gate * (_lin(ce_n, p['lin_self']['w'], p['lin_self']['b']) - agg)
    center = center + _lin(upd, p['out_proj']['w'], p['out_proj']['b'])
    h = _ln(center, p['norm2']['g'], p['norm2']['b'])
    h = jax.nn.relu(_lin(h, p['mlp']['w1'], p['mlp']['b1']))
    center = center + _lin(h, p['mlp']['w2'], p['mlp']['b2'])
    return center


# speedup vs baseline: 22.5720x; 1.5048x over previous
import functools

import jax
import jax.numpy as jnp
from jax import lax
from jax.experimental import pallas as pl
from jax.experimental.pallas import tpu as pltpu
from jax.experimental.pallas import tpu_sc as plsc

N = 50000
E = 800000
EMBED = 64
HEADS = 8
HEAD_DIM = EMBED // HEADS

_NW = 32
_PER_W = E // _NW
_GCH = 200


def _sc_gather(tab, src, dst):
    mesh = plsc.VectorSubcoreMesh(core_axis_name="c", subcore_axis_name="s")

    @functools.partial(
        pl.kernel,
        out_type=(jax.ShapeDtypeStruct((E, 128), jnp.float32),
                  jax.ShapeDtypeStruct((E, 128), jnp.float32)),
        mesh=mesh,
        scratch_types=[
            pltpu.VMEM((_GCH,), jnp.int32),
            pltpu.VMEM((_GCH,), jnp.int32),
            pltpu.VMEM((_GCH, 128), jnp.float32),
            pltpu.VMEM((_GCH, 128), jnp.float32),
            pltpu.SemaphoreType.DMA,
            pltpu.SemaphoreType.DMA,
        ],
    )
    def k(tab_hbm, src_hbm, dst_hbm, os_hbm, od_hbm, si, di, bb, qb, s0, s1):
        wid = lax.axis_index("s") * 2 + lax.axis_index("c")
        base = wid * _PER_W

        @pl.loop(0, _PER_W // _GCH)
        def _(i):
            off = base + i * _GCH
            pltpu.sync_copy(src_hbm.at[pl.ds(off, _GCH)], si)
            pltpu.sync_copy(dst_hbm.at[pl.ds(off, _GCH)], di)
            cb = pltpu.async_copy(tab_hbm.at[si], bb, s0)
            cq = pltpu.async_copy(tab_hbm.at[di], qb, s1)
            cb.wait()
            cq.wait()
            pltpu.sync_copy(bb, os_hbm.at[pl.ds(off, _GCH)])
            pltpu.sync_copy(qb, od_hbm.at[pl.ds(off, _GCH)])

    return k(tab, src, dst)


_SCH = 128
_SE_A = 50048
_SE_LAST = E - 15 * _SE_A
_Q = 12512
_AR = 12800


def _sc_scatter(payload, dst, zeros128):
    mesh = plsc.VectorSubcoreMesh(core_axis_name="c", subcore_axis_name="s")

    @functools.partial(
        pl.kernel,
        out_type=jax.ShapeDtypeStruct((4 * _Q, 128), jnp.float32),
        mesh=mesh,
        scratch_types=[
            pltpu.VMEM((_SCH, 128), jnp.float32),
            pltpu.VMEM((_SCH,), jnp.int32),
            pltpu.VMEM_SHARED((_AR, 128), jnp.float32),
            pltpu.SemaphoreType.DMA,
        ],
    )
    def k(p_hbm, dst_hbm, z_hbm, o_hbm, pay, idx, acc, sem):
        c = lax.axis_index("c")
        s = lax.axis_index("s")
        trash = _Q + s
        ebase = s * _SE_A

        for p in range(2):
            qbase = (2 * p + c) * _Q
            @pl.loop(0, 5)
            def _(j):
                pltpu.sync_copy(z_hbm, acc.at[pl.ds(s * 800 + j * 160, 160)])

            plsc.subcore_barrier()

            def run(nch):
                @pl.loop(0, nch)
                def _(i):
                    off = ebase + i * _SCH
                    pltpu.sync_copy(dst_hbm.at[pl.ds(off, _SCH)], idx)
                    pltpu.sync_copy(p_hbm.at[pl.ds(off, _SCH)], pay)

                    @pl.loop(0, _SCH // 16)
                    def _(j):
                        d = idx[pl.ds(j * 16, 16)] - qbase
                        ok = (d >= 0) & (d < _Q)
                        idx[pl.ds(j * 16, 16)] = jnp.where(ok, d, trash)

                    pltpu.sync_copy(pay, acc.at[idx], add=True)

            @pl.when(s < 15)
            def _():
                run(_SE_A // _SCH)

            @pl.when(s == 15)
            def _():
                run(_SE_LAST // _SCH)

            plsc.subcore_barrier()
            @pl.when(s < 15)
            def _():
                pltpu.sync_copy(acc.at[pl.ds(s * 784, 784)],
                                o_hbm.at[pl.ds(qbase + s * 784, 784)])

            @pl.when(s == 15)
            def _():
                pltpu.sync_copy(acc.at[pl.ds(15 * 784, _Q - 15 * 784)],
                                o_hbm.at[pl.ds(qbase + 15 * 784, _Q - 15 * 784)])

            plsc.subcore_barrier()

    return k(payload, dst, zeros128)


_BE = 2000


def _edge_dense(ea, os_, od_, w, sel, selt):
    (b1w1, b1b1, b1g1, b1be1, b1w2, b1b2,
     ag1, abe1, aw, ab, ag2, abe2, kw, kb, vw, vb) = w

    def body(ea_r, os_r, od_r, b1w1_r, b1b1_r, b1g1_r, b1be1_r, b1w2_r,
             b1b2_r, ag1_r, abe1_r, aw_r, ab_r, ag2_r, abe2_r, kw_r, kb_r,
             vw_r, vb_r, sel_r, selt_r, out_r):
        h1 = jax.nn.relu(_ln(ea_r[...] @ b1w1_r[...] + b1b1_r[...],
                             b1g1_r[...], b1be1_r[...]))
        s = os_r[:, :64] + h1 @ b1w2_r[...] + b1b2_r[...]
        hn = jax.nn.relu(_ln(s, ag1_r[...], abe1_r[...]))
        nbr = _ln(hn @ aw_r[...] + ab_r[...], ag2_r[...], abe2_r[...])
        k = nbr @ kw_r[...] + kb_r[...]
        v = nbr @ vw_r[...] + vb_r[...]
        q = od_r[:, 64:]
        ex = jnp.exp((q * k) @ sel_r[...])
        evx = v * (ex @ selt_r[...])
        out_r[:, 0:8] = ex
        out_r[:, 8:72] = evx
        out_r[:, 72:128] = jnp.zeros((_BE, 56), jnp.float32)

    full = lambda shp: pl.BlockSpec(shp, lambda i: (0, 0))
    blk = lambda c: pl.BlockSpec((_BE, c), lambda i: (i, 0))
    return pl.pallas_call(
        body,
        grid=(E // _BE,),
        in_specs=[blk(2), blk(128), blk(128),
                  full((2, 64)), full((1, 64)), full((1, 64)), full((1, 64)),
                  full((64, 64)), full((1, 64)),
                  full((1, 64)), full((1, 64)), full((64, 64)), full((1, 64)),
                  full((1, 64)), full((1, 64)),
                  full((64, 64)), full((1, 64)), full((64, 64)), full((1, 64)),
                  full((64, 8)), full((8, 64))],
        out_specs=blk(128),
        out_shape=jax.ShapeDtypeStruct((E, 128), jnp.float32),
    )(ea, os_, od_, b1w1, b1b1.reshape(1, 64), b1g1.reshape(1, 64),
      b1be1.reshape(1, 64), b1w2, b1b2.reshape(1, 64), ag1.reshape(1, 64),
      abe1.reshape(1, 64), aw, ab.reshape(1, 64), ag2.reshape(1, 64),
      abe2.reshape(1, 64), kw, kb.reshape(1, 64), vw, vb.reshape(1, 64),
      sel, selt)


def _ln(x, g, b, eps=1e-5):
    m = x.mean(-1, keepdims=True)
    v = ((x - m) ** 2).mean(-1, keepdims=True)
    return (x - m) / jnp.sqrt(v + eps) * g + b


def _lin(x, w, b):
    return x @ w + b


def kernel(x, t, edge_index, edge_attr, bos_mask, params):
    p = params
    cp = p['center']
    h = jax.nn.relu(_ln(_lin(x, cp['w1'], cp['b1']), cp['g1'], cp['be1']))
    h = jax.nn.relu(_ln(_lin(h, cp['w2'], cp['b2']), cp['g2'], cp['be2']))
    center = _ln(_lin(h, cp['w3'], cp['b3']), cp['g3'], cp['be3'])
    center = jnp.where(bos_mask[:, None], p['bos_token'][t], center)
    ce_n = _ln(center, p['norm1']['g'], p['norm1']['b'])
    qn = _lin(ce_n, p['lin_q']['w'], p['lin_q']['b'])
    b0 = p['nbr']['branch0']
    hb = jax.nn.relu(_ln(_lin(x, b0['w1'], b0['b1']), b0['g1'], b0['be1']))
    b0n = _lin(hb, b0['w2'], b0['b2'])

    src = edge_index[0]
    dst = edge_index[1]
    b1 = p['nbr']['branch1']
    a = p['nbr']['aggr']
    tab = jnp.concatenate([b0n, qn], axis=1)
    os_, od_ = _sc_gather(tab, src, dst)
    sel = jnp.repeat(jnp.eye(8, dtype=jnp.float32), HEAD_DIM, axis=0)
    w = (b1['w1'], b1['b1'], b1['g1'], b1['be1'], b1['w2'], b1['b2'],
         a['g1'], a['be1'], a['w'], a['b'], a['g2'], a['be2'],
         p['lin_k']['w'], p['lin_k']['b'], p['lin_v']['w'], p['lin_v']['b'])
    payload = _edge_dense(edge_attr, os_, od_, w,
                          sel / (HEAD_DIM ** 0.5), sel.T)
    zeros128 = jnp.zeros((160, 128), jnp.float32)
    o = _sc_scatter(payload, dst, zeros128)
    denom = o[:N, :8]
    num = o[:N, 8:72]
    agg = num / jnp.repeat(denom + 1e-16, HEAD_DIM, axis=-1)

    gate = jax.nn.sigmoid(_lin(agg, p['lin_ih']['w'], p['lin_ih']['b']) +
                          _lin(ce_n, p['lin_hh']['w'], p['lin_hh']['b']))
    upd = agg + gate * (_lin(ce_n, p['lin_self']['w'], p['lin_self']['b']) - agg)
    center = center + _lin(upd, p['out_proj']['w'], p['out_proj']['b'])
    h = _ln(center, p['norm2']['g'], p['norm2']['b'])
    h = jax.nn.relu(_lin(h, p['mlp']['w1'], p['mlp']['b1']))
    center = center + _lin(h, p['mlp']['w2'], p['mlp']['b2'])
    return center
